# baseline (speedup 1.0000x reference)
import jax
import jax.numpy as jnp
from jax.experimental import pallas as pl

_B, _N, _G = 8, 4096, 128
_C = 8
_FH, _FW = 8, 8
_ZW, _ZH = 1.1, 1.1
_BLK = 256
_NBLK = _N // _BLK


def _body(bt_ref, gt_ref, ids_ref, flg_ref, lab_ref, off_ref, mask_ref):
    bt = bt_ref[0]
    gtt = gt_ref[0]
    idx = ids_ref[0]
    flg = flg_ref[0]

    oh = (jax.lax.broadcasted_iota(jnp.int32, (_G, _BLK), 0) == idx)
    ga = jnp.dot(gtt, oh.astype(jnp.float32),
                 preferred_element_type=jnp.float32,
                 precision=jax.lax.Precision.HIGHEST)

    x1, y1, x2, y2 = bt[0:1], bt[1:2], bt[2:3], bt[3:4]
    cx = (x1 + x2) / 2.0
    cy = (y1 + y2) / 2.0
    w = (x2 - x1) * _ZW
    h = (y2 - y1) * _ZH
    ax1 = cx - w / 2.0
    ay1 = cy - h / 2.0
    ax2 = cx + w / 2.0
    ay2 = cy + h / 2.0

    gx1, gy1, gx2, gy2, lbl = ga[0:1], ga[1:2], ga[2:3], ga[3:4], ga[4:5]
    rx1 = gx1 - ax1
    ry1 = gy1 - ay1
    rx2 = gx2 - ax1
    ry2 = gy2 - ay1
    rw = rx2 - rx1
    rh = ry2 - ry1
    rcx = (rx1 + rx2) / 2.0
    rcy = (ry1 + ry2) / 2.0
    sw = (ax2 - ax1) / _FW
    sh = (ay2 - ay1) / _FH
    w_sigma = rw / 2.0 / sw
    h_sigma = rh / 2.0 / sh
    pw = rcx / sw
    ph = rcy / sh
    q1 = rx1 / sw
    q2 = ry1 / sh
    q3 = rx2 / sw
    q4 = ry2 / sh
    flgf = flg.astype(jnp.float32)

    rows = jnp.concatenate(
        [pw, ph, w_sigma, h_sigma, q1, q2, q3, q4, lbl, flgf], axis=0)
    cols = jnp.transpose(rows)
    pwc, phc = cols[:, 0:1], cols[:, 1:2]
    wsc, hsc = cols[:, 2:3], cols[:, 3:4]
    q1c, q2c, q3c, q4c = cols[:, 4:5], cols[:, 5:6], cols[:, 6:7], cols[:, 7:8]
    lblc, flgc = cols[:, 8:9], cols[:, 9:10]

    def e3(v):
        return v[:, :, None]

    ind_w = jax.lax.broadcasted_iota(jnp.int32, (_BLK, _FH, _FW), 2).astype(jnp.float32)
    ind_h = jax.lax.broadcasted_iota(jnp.int32, (_BLK, _FH, _FW), 1).astype(jnp.float32)
    w_term = jnp.square((e3(pwc) - ind_w - 0.5) / e3(wsc))
    h_term = jnp.square((e3(phc) - ind_h - 0.5) / e3(hsc))
    g = jnp.exp(-(w_term + h_term))
    cond = ((jnp.abs(ind_w + 0.5 - e3(pwc)) < e3(wsc))
            & (jnp.abs(ind_h + 0.5 - e3(phc)) < e3(hsc)))
    g = jnp.where(cond, g, 0.0)
    lab_ref[...] = jnp.broadcast_to(g[:, None], (_BLK, _C, _FH, _FW))

    ox1 = e3(q1c) - (ind_w + 0.5)
    oy1 = e3(q2c) - (ind_h + 0.5)
    ox2 = e3(q3c) - (ind_w + 0.5)
    oy2 = e3(q4c) - (ind_h + 0.5)
    off_ref[...] = jnp.stack([ox1, oy1, ox2, oy2], axis=1)

    cls = jax.lax.broadcasted_iota(jnp.int32, (_BLK, _C), 1).astype(jnp.float32)
    pos = flgc > 0.0
    nn = jnp.where(flgc != 0.0, lblc, 0.0)
    mone = pos & (nn > 0.0)
    m = (cls == (jnp.abs(lblc) - 1.0)) & mone
    mask_ref[...] = m.astype(jnp.int8)


def kernel(boxes, gt_boxes, match_pos_flag, match_gt_id):
    bt = boxes.transpose(0, 2, 1)
    gtt = jnp.pad(gt_boxes, ((0, 0), (0, 0), (0, 3))).transpose(0, 2, 1)
    ids = match_gt_id.astype(jnp.int32).reshape(_B * _NBLK, 1, _BLK)
    flg = match_pos_flag.astype(jnp.int32).reshape(_B * _NBLK, 1, _BLK)
    lab, off, mask8 = pl.pallas_call(
        _body,
        grid=(_B, _NBLK),
        in_specs=[
            pl.BlockSpec((1, 4, _BLK), lambda b, j: (b, 0, j)),
            pl.BlockSpec((1, 8, _G), lambda b, j: (b, 0, 0)),
            pl.BlockSpec((1, 1, _BLK), lambda b, j: (b * _NBLK + j, 0, 0)),
            pl.BlockSpec((1, 1, _BLK), lambda b, j: (b * _NBLK + j, 0, 0)),
        ],
        out_specs=[
            pl.BlockSpec((_BLK, _C, _FH, _FW),
                         lambda b, j: (b * _NBLK + j, 0, 0, 0)),
            pl.BlockSpec((_BLK, 4, _FH, _FW),
                         lambda b, j: (b * _NBLK + j, 0, 0, 0)),
            pl.BlockSpec((_BLK, _C), lambda b, j: (b * _NBLK + j, 0)),
        ],
        out_shape=[
            jax.ShapeDtypeStruct((_B * _N, _C, _FH, _FW), jnp.float32),
            jax.ShapeDtypeStruct((_B * _N, 4, _FH, _FW), jnp.float32),
            jax.ShapeDtypeStruct((_B * _N, _C), jnp.int8),
        ],
    )(bt, gtt, ids, flg)
    return lab, off, mask8.astype(bool)
